# baseline (speedup 1.0000x reference)
import functools

import jax
import jax.numpy as jnp
from jax import lax
from jax.experimental import pallas as pl
from jax.experimental.pallas import tpu as pltpu
from jax.experimental.pallas import tpu_sc as plsc

K = 20
N = 2048
B_ROWS = 32768
NC, NS, L = 2, 16, 16
NW = NC * NS
CHUNK_ROWS = 8
SC_ROWS = 23552
TC_ROWS = B_ROWS - SC_ROWS
ROWS_PER_W = SC_ROWS // NW
CHUNKS_PER_W = ROWS_PER_W // CHUNK_ROWS
CHUNK_ELEMS = CHUNK_ROWS * N
OUT_PER_W = ROWS_PER_W * K
TC_BLOCK = 256

_INF = float("inf")


def _lane():
    return lax.iota(jnp.int32, L)


def _vsort(v):
    w, _ = plsc.sort_key_val(v, _lane())
    return w


def _exact_sort16(v, i):
    w, j = plsc.sort_key_val(v, i)
    occ, _ = plsc.scan_count(w)
    rs = _lane() - occ
    key2 = rs * jnp.int32(2048) + j
    _, j2 = plsc.sort_key_val(key2, j)
    return w, j2


def _lex_take(av, ai, bv, bi):
    return (av < bv) | ((av == bv) & (ai < bi))


def _merge32(a, b):
    rb = lax.rev(b, (0,))
    lo = jnp.minimum(a, rb)
    hi = jnp.maximum(a, rb)
    return _vsort(lo), _vsort(hi)


def _threshold(a0, a1, a2, a3):
    p0, p1 = _merge32(_vsort(a0), _vsort(a1))
    q0, q1 = _merge32(_vsort(a2), _vsort(a3))
    l0 = jnp.minimum(p0, lax.rev(q1, (0,)))
    l1 = jnp.minimum(p1, lax.rev(q0, (0,)))
    h = _vsort(jnp.maximum(l0, l1))
    return jnp.min(jnp.where(_lane() >= 4, h, _INF))


def _scalar(v16):
    return lax.squeeze(lax.slice(v16, (0,), (1,)), (0,))


R_IL = 4
HALF = N // 2


def _group_body(xb, cands, ob, p, rw_base):
    R = R_IL
    rows = [R * p + k for k in range(R)]

    inf16 = jnp.full((L,), _INF)

    def make_ph1(pair):
        def ph1(i, acc):
            a = list(acc)
            for kk, k in enumerate(pair):
                o = 64 * i
                for q in range(4):
                    a[4 * kk + q] = jnp.minimum(
                        a[4 * kk + q], xb[rows[k], pl.ds(o + 16 * q, L)])
            return tuple(a)
        return ph1

    acc = []
    for h in range(R // 2):
        acc.extend(lax.fori_loop(
            0, 32, make_ph1((2 * h, 2 * h + 1)), (inf16,) * 8))
    tvs = [jnp.full((L,), _threshold(*acc[4 * k:4 * k + 4])) for k in range(R)]

    def make_ph2(pair):
        def ph2(i, carry):
            ca = list(carry[:2])
            cb = list(carry[2:4])
            idxv = carry[4]
            for q in range(2):
                iq = idxv + jnp.int32(16 * q)
                for kk, k in enumerate(pair):
                    va = xb[rows[k], pl.ds(32 * i + 16 * q, L)]
                    vb = xb[rows[k], pl.ds(HALF + 32 * i + 16 * q, L)]
                    ma = va <= tvs[k]
                    mb = vb <= tvs[k]
                    plsc.store_compressed(cands[k].at[pl.ds(ca[kk], L)], iq,
                                          mask=ma)
                    plsc.store_compressed(
                        cands[k].at[pl.ds(jnp.int32(HALF) + cb[kk], L)],
                        iq + jnp.int32(HALF), mask=mb)
                    ca[kk] = ca[kk] + _scalar(
                        plsc.all_reduce_population_count(ma))
                    cb[kk] = cb[kk] + _scalar(
                        plsc.all_reduce_population_count(mb))
            return tuple(ca) + tuple(cb) + (idxv + jnp.int32(32),)
        return ph2

    ca = [None] * R
    cb = [None] * R
    for h in range(R // 2):
        carry = lax.fori_loop(
            0, HALF // 32, make_ph2((2 * h, 2 * h + 1)),
            (jnp.int32(0),) * 4 + (_lane(),))
        ca[2 * h], ca[2 * h + 1] = carry[0], carry[1]
        cb[2 * h], cb[2 * h + 1] = carry[2], carry[3]

    zero16 = jnp.zeros((L,), jnp.int32)
    nva = [(c + jnp.int32(15)) >> 4 for c in ca]
    nvb = [(c + jnp.int32(15)) >> 4 for c in cb]
    nv = nva[0] + nvb[0]
    for k in range(1, R):
        nv = jnp.maximum(nv, nva[k] + nvb[k])

    def merge_step(sv0, sv1, si0, si1, k, jv):
        in_a = jv < nva[k]
        jb = jv - nva[k]
        roff = jnp.where(in_a, 16 * jv, jnp.int32(HALF) + 16 * jb)
        roff = jnp.minimum(roff, jnp.int32(2 * HALF))
        lp = _lane() + jnp.where(in_a, 16 * jv, 16 * jb)
        m = lp < jnp.where(in_a, ca[k], cb[k])
        idx = jnp.where(m, cands[k][pl.ds(roff, L)], 0)
        rsplat = jnp.full((L,), rows[k], jnp.int32)
        vals = plsc.load_gather(xb, [rsplat, idx])
        vals = jnp.where(m, vals, _INF)
        w, j = _exact_sort16(vals, idx)
        rw_, rj_ = lax.rev(w, (0,)), lax.rev(j, (0,))
        take = _lex_take(sv1, si1, rw_, rj_)
        l1v = jnp.where(take, sv1, rw_)
        l1i = jnp.where(take, si1, rj_)
        take2 = _lex_take(sv0, si0, l1v, l1i)
        lv = jnp.where(take2, sv0, l1v)
        li = jnp.where(take2, si0, l1i)
        hv = jnp.where(take2, l1v, sv0)
        hi = jnp.where(take2, l1i, si0)
        sv0, si0 = _exact_sort16(lv, li)
        sv1, si1 = _exact_sort16(hv, hi)
        return sv0, sv1, si0, si1

    def ph3(jv, carry):
        out = []
        for k in range(R):
            out.append(merge_step(*carry[4 * k:4 * k + 4], k, jv))
        return tuple(x for s in out for x in s)

    init = (inf16, inf16, zero16, zero16) * R
    res = lax.fori_loop(0, nv, ph3, init)

    lo_mask = _lane() >= 1
    hi_mask = _lane() < 5
    for k in range(R):
        si0, si1 = res[4 * k + 2], res[4 * k + 3]
        o = 20 * (rw_base + R * p + k)
        plsc.store_compressed(ob.at[pl.ds(o, L)], si0, mask=lo_mask)
        plsc.store_compressed(ob.at[pl.ds(o + 15, L)], si1, mask=hi_mask)


def _process_chunk(xb, cands, ob, g):
    rw_base = CHUNK_ROWS * g

    def groupf(p, _):
        _group_body(xb, cands, ob, p, rw_base)
        return 0
    lax.fori_loop(0, CHUNK_ROWS // R_IL, groupf, 0)


_mesh = plsc.VectorSubcoreMesh(
    core_axis_name="c", subcore_axis_name="s", num_cores=NC, num_subcores=NS)


@functools.partial(
    pl.kernel,
    out_type=jax.ShapeDtypeStruct((SC_ROWS * K,), jnp.int32),
    mesh=_mesh,
    scratch_types=[
        pltpu.VMEM((CHUNK_ROWS, N), jnp.float32),
        pltpu.VMEM((CHUNK_ROWS, N), jnp.float32),
        pltpu.VMEM((OUT_PER_W + 16,), jnp.int32),
        pltpu.VMEM((N + 16,), jnp.int32),
        pltpu.VMEM((N + 16,), jnp.int32),
        pltpu.VMEM((N + 16,), jnp.int32),
        pltpu.VMEM((N + 16,), jnp.int32),
        pltpu.SemaphoreType.DMA,
        pltpu.SemaphoreType.DMA,
    ],
    compiler_params=pltpu.CompilerParams(
        needs_layout_passes=False, use_tc_tiling_on_sc=True),
)
def _sc_topk(x_hbm, o_hbm, xb0, xb1, ob, cand0, cand1, cand2, cand3,
             sem0, sem1):
    cands = [cand0, cand1, cand2, cand3]
    wid = lax.axis_index("s") * NC + lax.axis_index("c")
    row0 = wid * ROWS_PER_W

    def chunk_src(g):
        gc = jnp.minimum(g, CHUNKS_PER_W - 1)
        return x_hbm.at[pl.ds(row0 + CHUNK_ROWS * gc, CHUNK_ROWS), :]

    pltpu.async_copy(chunk_src(jnp.int32(0)), xb0, sem0)
    pltpu.async_copy(chunk_src(jnp.int32(1)), xb1, sem1)

    def pair(g2, _):
        g = 2 * g2
        pltpu.make_async_copy(chunk_src(g), xb0, sem0).wait()
        _process_chunk(xb0, cands, ob, g)
        pltpu.async_copy(chunk_src(g + 2), xb0, sem0)
        pltpu.make_async_copy(chunk_src(g + 1), xb1, sem1).wait()
        _process_chunk(xb1, cands, ob, g + 1)
        pltpu.async_copy(chunk_src(g + 3), xb1, sem1)
        return 0

    lax.fori_loop(0, CHUNKS_PER_W // 2, pair, 0)
    pltpu.make_async_copy(chunk_src(jnp.int32(0)), xb0, sem0).wait()
    pltpu.make_async_copy(chunk_src(jnp.int32(0)), xb1, sem1).wait()

    pltpu.sync_copy(ob.at[pl.ds(0, OUT_PER_W)],
                    o_hbm.at[pl.ds(wid * OUT_PER_W, OUT_PER_W)])


def _tc_topk_body(x_ref, o_ref):
    x = x_ref[...]
    r = x.shape[0]
    iota = lax.broadcasted_iota(jnp.int32, (r, N), 1)
    cols = []
    for t in range(K + 1):
        m = jnp.min(x, axis=1, keepdims=True)
        idx = jnp.min(jnp.where(x == m, iota, N), axis=1, keepdims=True)
        if t > 0:
            cols.append(idx)
        x = jnp.where(iota == idx, jnp.float32(jnp.inf), x)
    o_ref[...] = jnp.concatenate(cols, axis=1)


@jax.jit
def kernel(inputs):
    d = inputs
    b, q, n = d.shape
    rows2d = d.reshape(b * q, n)
    sc_out = _sc_topk(rows2d).reshape(SC_ROWS, K)
    tc_out = pl.pallas_call(
        _tc_topk_body,
        grid=(TC_ROWS // TC_BLOCK,),
        in_specs=[pl.BlockSpec((TC_BLOCK, N),
                               lambda i: (i + SC_ROWS // TC_BLOCK, 0))],
        out_specs=pl.BlockSpec((TC_BLOCK, K), lambda i: (i, 0)),
        out_shape=jax.ShapeDtypeStruct((TC_ROWS, K), jnp.int32),
        compiler_params=pltpu.CompilerParams(
            dimension_semantics=("arbitrary",),
        ),
    )(rows2d)
    return jnp.concatenate([sc_out, tc_out], axis=0).reshape(b, q, K)


# speedup vs baseline: 1.0213x; 1.0213x over previous
import functools

import jax
import jax.numpy as jnp
from jax import lax
from jax.experimental import pallas as pl
from jax.experimental.pallas import tpu as pltpu
from jax.experimental.pallas import tpu_sc as plsc

K = 20
N = 2048
B_ROWS = 32768
NC, NS, L = 2, 16, 16
NW = NC * NS
CHUNK_ROWS = 8
SC_ROWS = 23040
TC_ROWS = B_ROWS - SC_ROWS
ROWS_PER_W = SC_ROWS // NW
CHUNKS_PER_W = ROWS_PER_W // CHUNK_ROWS
CHUNK_ELEMS = CHUNK_ROWS * N
OUT_PER_W = ROWS_PER_W * K
TC_BLOCK = 256

_INF = float("inf")


def _lane():
    return lax.iota(jnp.int32, L)


def _vsort(v):
    w, _ = plsc.sort_key_val(v, _lane())
    return w


def _exact_sort16(v, i):
    w, j = plsc.sort_key_val(v, i)
    occ, _ = plsc.scan_count(w)
    rs = _lane() - occ
    key2 = rs * jnp.int32(2048) + j
    _, j2 = plsc.sort_key_val(key2, j)
    return w, j2


def _lex_take(av, ai, bv, bi):
    return (av < bv) | ((av == bv) & (ai < bi))


def _merge32(a, b):
    rb = lax.rev(b, (0,))
    lo = jnp.minimum(a, rb)
    hi = jnp.maximum(a, rb)
    return _vsort(lo), _vsort(hi)


def _threshold(a0, a1, a2, a3):
    p0, p1 = _merge32(_vsort(a0), _vsort(a1))
    q0, q1 = _merge32(_vsort(a2), _vsort(a3))
    l0 = jnp.minimum(p0, lax.rev(q1, (0,)))
    l1 = jnp.minimum(p1, lax.rev(q0, (0,)))
    h = _vsort(jnp.maximum(l0, l1))
    return jnp.min(jnp.where(_lane() >= 4, h, _INF))


def _scalar(v16):
    return lax.squeeze(lax.slice(v16, (0,), (1,)), (0,))


R_IL = 4
HALF = N // 2


def _group_body(xb, cands, ob, p, rw_base):
    R = R_IL
    rows = [R * p + k for k in range(R)]

    inf16 = jnp.full((L,), _INF)

    def make_ph1(pair):
        def ph1(i, acc):
            a = list(acc)
            for kk, k in enumerate(pair):
                o = 64 * i
                for q in range(4):
                    a[4 * kk + q] = jnp.minimum(
                        a[4 * kk + q], xb[rows[k], pl.ds(o + 16 * q, L)])
            return tuple(a)
        return ph1

    acc = []
    for h in range(R // 2):
        acc.extend(lax.fori_loop(
            0, 32, make_ph1((2 * h, 2 * h + 1)), (inf16,) * 8))
    tvs = [jnp.full((L,), _threshold(*acc[4 * k:4 * k + 4])) for k in range(R)]

    def make_ph2(pair):
        def ph2(i, carry):
            ca = list(carry[:2])
            cb = list(carry[2:4])
            idxv = carry[4]
            for q in range(2):
                iq = idxv + jnp.int32(16 * q)
                for kk, k in enumerate(pair):
                    va = xb[rows[k], pl.ds(32 * i + 16 * q, L)]
                    vb = xb[rows[k], pl.ds(HALF + 32 * i + 16 * q, L)]
                    ma = va <= tvs[k]
                    mb = vb <= tvs[k]
                    plsc.store_compressed(cands[k].at[pl.ds(ca[kk], L)], iq,
                                          mask=ma)
                    plsc.store_compressed(
                        cands[k].at[pl.ds(jnp.int32(HALF) + cb[kk], L)],
                        iq + jnp.int32(HALF), mask=mb)
                    ca[kk] = ca[kk] + _scalar(
                        plsc.all_reduce_population_count(ma))
                    cb[kk] = cb[kk] + _scalar(
                        plsc.all_reduce_population_count(mb))
            return tuple(ca) + tuple(cb) + (idxv + jnp.int32(32),)
        return ph2

    ca = [None] * R
    cb = [None] * R
    for h in range(R // 2):
        carry = lax.fori_loop(
            0, HALF // 32, make_ph2((2 * h, 2 * h + 1)),
            (jnp.int32(0),) * 4 + (_lane(),))
        ca[2 * h], ca[2 * h + 1] = carry[0], carry[1]
        cb[2 * h], cb[2 * h + 1] = carry[2], carry[3]

    zero16 = jnp.zeros((L,), jnp.int32)
    nva = [(c + jnp.int32(15)) >> 4 for c in ca]
    nvb = [(c + jnp.int32(15)) >> 4 for c in cb]
    nv = nva[0] + nvb[0]
    for k in range(1, R):
        nv = jnp.maximum(nv, nva[k] + nvb[k])

    def merge_step(sv0, sv1, si0, si1, k, jv):
        in_a = jv < nva[k]
        jb = jv - nva[k]
        roff = jnp.where(in_a, 16 * jv, jnp.int32(HALF) + 16 * jb)
        roff = jnp.minimum(roff, jnp.int32(2 * HALF))
        lp = _lane() + jnp.where(in_a, 16 * jv, 16 * jb)
        m = lp < jnp.where(in_a, ca[k], cb[k])
        idx = jnp.where(m, cands[k][pl.ds(roff, L)], 0)
        rsplat = jnp.full((L,), rows[k], jnp.int32)
        vals = plsc.load_gather(xb, [rsplat, idx])
        vals = jnp.where(m, vals, _INF)
        w, j = _exact_sort16(vals, idx)
        rw_, rj_ = lax.rev(w, (0,)), lax.rev(j, (0,))
        take = _lex_take(sv1, si1, rw_, rj_)
        l1v = jnp.where(take, sv1, rw_)
        l1i = jnp.where(take, si1, rj_)
        take2 = _lex_take(sv0, si0, l1v, l1i)
        lv = jnp.where(take2, sv0, l1v)
        li = jnp.where(take2, si0, l1i)
        hv = jnp.where(take2, l1v, sv0)
        hi = jnp.where(take2, l1i, si0)
        sv0, si0 = _exact_sort16(lv, li)
        sv1, si1 = _exact_sort16(hv, hi)
        return sv0, sv1, si0, si1

    def ph3(jv, carry):
        out = []
        for k in range(R):
            out.append(merge_step(*carry[4 * k:4 * k + 4], k, jv))
        return tuple(x for s in out for x in s)

    init = (inf16, inf16, zero16, zero16) * R
    res = lax.fori_loop(0, nv, ph3, init)

    lo_mask = _lane() >= 1
    hi_mask = _lane() < 5
    for k in range(R):
        si0, si1 = res[4 * k + 2], res[4 * k + 3]
        o = 20 * (rw_base + R * p + k)
        plsc.store_compressed(ob.at[pl.ds(o, L)], si0, mask=lo_mask)
        plsc.store_compressed(ob.at[pl.ds(o + 15, L)], si1, mask=hi_mask)


def _process_chunk(xb, cands, ob, g):
    rw_base = CHUNK_ROWS * g

    def groupf(p, _):
        _group_body(xb, cands, ob, p, rw_base)
        return 0
    lax.fori_loop(0, CHUNK_ROWS // R_IL, groupf, 0)


_mesh = plsc.VectorSubcoreMesh(
    core_axis_name="c", subcore_axis_name="s", num_cores=NC, num_subcores=NS)


@functools.partial(
    pl.kernel,
    out_type=jax.ShapeDtypeStruct((SC_ROWS * K,), jnp.int32),
    mesh=_mesh,
    scratch_types=[
        pltpu.VMEM((CHUNK_ROWS, N), jnp.float32),
        pltpu.VMEM((CHUNK_ROWS, N), jnp.float32),
        pltpu.VMEM((OUT_PER_W + 16,), jnp.int32),
        pltpu.VMEM((N + 16,), jnp.int32),
        pltpu.VMEM((N + 16,), jnp.int32),
        pltpu.VMEM((N + 16,), jnp.int32),
        pltpu.VMEM((N + 16,), jnp.int32),
        pltpu.SemaphoreType.DMA,
        pltpu.SemaphoreType.DMA,
    ],
    compiler_params=pltpu.CompilerParams(
        needs_layout_passes=False, use_tc_tiling_on_sc=True),
)
def _sc_topk(x_hbm, o_hbm, xb0, xb1, ob, cand0, cand1, cand2, cand3,
             sem0, sem1):
    cands = [cand0, cand1, cand2, cand3]
    wid = lax.axis_index("s") * NC + lax.axis_index("c")
    row0 = wid * ROWS_PER_W

    def chunk_src(g):
        gc = jnp.minimum(g, CHUNKS_PER_W - 1)
        return x_hbm.at[pl.ds(row0 + CHUNK_ROWS * gc, CHUNK_ROWS), :]

    pltpu.async_copy(chunk_src(jnp.int32(0)), xb0, sem0)
    pltpu.async_copy(chunk_src(jnp.int32(1)), xb1, sem1)

    def pair(g2, _):
        g = 2 * g2
        pltpu.make_async_copy(chunk_src(g), xb0, sem0).wait()
        _process_chunk(xb0, cands, ob, g)
        pltpu.async_copy(chunk_src(g + 2), xb0, sem0)
        pltpu.make_async_copy(chunk_src(g + 1), xb1, sem1).wait()
        _process_chunk(xb1, cands, ob, g + 1)
        pltpu.async_copy(chunk_src(g + 3), xb1, sem1)
        return 0

    lax.fori_loop(0, CHUNKS_PER_W // 2, pair, 0)
    pltpu.make_async_copy(chunk_src(jnp.int32(0)), xb0, sem0).wait()
    pltpu.make_async_copy(chunk_src(jnp.int32(0)), xb1, sem1).wait()

    pltpu.sync_copy(ob.at[pl.ds(0, OUT_PER_W)],
                    o_hbm.at[pl.ds(wid * OUT_PER_W, OUT_PER_W)])


def _tc_topk_body(x_ref, o_ref):
    x = x_ref[...]
    r = x.shape[0]
    iota = lax.broadcasted_iota(jnp.int32, (r, N), 1)
    cols = []
    for t in range(K + 1):
        m = jnp.min(x, axis=1, keepdims=True)
        idx = jnp.min(jnp.where(x == m, iota, N), axis=1, keepdims=True)
        if t > 0:
            cols.append(idx)
        x = jnp.where(iota == idx, jnp.float32(jnp.inf), x)
    o_ref[...] = jnp.concatenate(cols, axis=1)


@jax.jit
def kernel(inputs):
    d = inputs
    b, q, n = d.shape
    rows2d = d.reshape(b * q, n)
    sc_out = _sc_topk(rows2d).reshape(SC_ROWS, K)
    tc_out = pl.pallas_call(
        _tc_topk_body,
        grid=(TC_ROWS // TC_BLOCK,),
        in_specs=[pl.BlockSpec((TC_BLOCK, N),
                               lambda i: (i + SC_ROWS // TC_BLOCK, 0))],
        out_specs=pl.BlockSpec((TC_BLOCK, K), lambda i: (i, 0)),
        out_shape=jax.ShapeDtypeStruct((TC_ROWS, K), jnp.int32),
        compiler_params=pltpu.CompilerParams(
            dimension_semantics=("arbitrary",),
        ),
    )(rows2d)
    return jnp.concatenate([sc_out, tc_out], axis=0).reshape(b, q, K)
